# baseline (speedup 1.0000x reference)
import functools

import jax
import jax.numpy as jnp
from jax import lax
from jax.experimental import pallas as pl
from jax.experimental.pallas import tpu as pltpu
from jax.experimental.pallas import tpu_sc as plsc


def _time_tc(t, time_w, time_b):
    E = t.shape[0]
    DT = time_w.shape[0]
    B = 1280
    assert E % B == 0
    grid = E // B

    def body(t_ref, w_ref, b_ref, o_ref):
        o_ref[...] = jnp.cos(t_ref[...] * w_ref[...] + b_ref[...])

    return pl.pallas_call(
        body,
        grid=(grid,),
        in_specs=[
            pl.BlockSpec((B, 1), lambda i: (i, 0)),
            pl.BlockSpec((1, DT), lambda i: (0, 0)),
            pl.BlockSpec((1, DT), lambda i: (0, 0)),
        ],
        out_specs=pl.BlockSpec((B, DT), lambda i: (i, 0)),
        out_shape=jax.ShapeDtypeStruct((E, DT), jnp.float32),
    )(t.reshape(E, 1), time_w.reshape(1, DT), time_b.reshape(1, DT))


def _sc_assemble(table, fidx2, tidx2, edge3, time3, out_w):
    CH, C = fidx2.shape
    N, D = table.shape
    DE = edge3.shape[-1]
    DT = time3.shape[-1]
    info = plsc.get_sparse_core_info()
    NC = info.num_cores
    NW = NC * info.num_subcores
    n_iter = -(-CH // NW)
    mesh = plsc.VectorSubcoreMesh(core_axis_name="c", subcore_axis_name="s")

    @functools.partial(
        pl.kernel,
        mesh=mesh,
        compiler_params=pltpu.CompilerParams(use_tc_tiling_on_sc=False),
        out_type=jax.ShapeDtypeStruct((CH, C, out_w), jnp.float32),
        scratch_types=[
            pltpu.VMEM((C,), jnp.int32),
            pltpu.VMEM((C,), jnp.int32),
            pltpu.VMEM((C, D), jnp.float32),
            pltpu.VMEM((C, D), jnp.float32),
            pltpu.SemaphoreType.DMA,
            pltpu.SemaphoreType.DMA,
        ],
    )
    def k(table_h, fidx_h, tidx_h, edge_h, time_h, out_h, fiv, tiv, fr, tr, s1, s2):
        wid = lax.axis_index("s") * NC + lax.axis_index("c")

        def body(g, carry):
            ch = g * NW + wid

            @pl.when(ch < CH)
            def _():
                pltpu.sync_copy(fidx_h.at[ch], fiv)
                pltpu.sync_copy(tidx_h.at[ch], tiv)
                cf = pltpu.async_copy(table_h.at[fiv], fr, s1)
                ct = pltpu.async_copy(table_h.at[tiv], tr, s2)
                cf.wait()
                ct.wait()
                pltpu.sync_copy(fr, out_h.at[ch, :, pl.ds(0, D)])
                pltpu.sync_copy(tr, out_h.at[ch, :, pl.ds(D + DE, D)])
                pltpu.sync_copy(edge_h.at[ch], out_h.at[ch, :, pl.ds(D, DE)])
                pltpu.sync_copy(time_h.at[ch], out_h.at[ch, :, pl.ds(D + DE + D, DT)])

            return carry

        lax.fori_loop(0, n_iter, body, 0)

    return k(table, fidx2, tidx2, edge3, time3)


def kernel(update_node_emb, edge_emb, from_idx, to_idx, t, time_w, time_b):
    N, D = update_node_emb.shape
    E, DE = edge_emb.shape
    DT = time_w.shape[0]
    out_w = D + DE + D + DT
    C = 128
    assert E % C == 0
    CH = E // C

    time_emb = _time_tc(t, time_w, time_b)
    fidx2 = from_idx.astype(jnp.int32).reshape(CH, C)
    tidx2 = to_idx.astype(jnp.int32).reshape(CH, C)
    edge3 = edge_emb.reshape(CH, C, DE)
    time3 = time_emb.reshape(CH, C, DT)
    out3 = _sc_assemble(update_node_emb, fidx2, tidx2, edge3, time3, out_w)
    return out3.reshape(E, out_w)
